# baseline (speedup 1.0000x reference)
import dataclasses

import jax
import jax.numpy as jnp
from jax import lax
from jax.experimental import pallas as pl
from jax.experimental.pallas import tpu as pltpu
from jax.experimental.pallas import tpu_sc as plsc

B = 16384
F = 26
K = 1000
VB = 16
GRP = 8
CHUNK = 16 * VB * GRP
NCH = B // CHUNK


def _gather_kernel(lab_hbm, mean_hbm, logvar_hbm, out_m_hbm, out_lv_hbm,
                   lab_v, mean_v, logvar_v, om_v, olv_v,
                   sem_a, sem_b, sem_c):
    wid = lax.axis_index("subcore") * 2 + lax.axis_index("core")

    @pl.when(wid < F)
    def _():
        col = pl.ds(wid, 1)
        ca = pltpu.async_copy(lab_hbm.at[col, :], lab_v, sem_a)
        cb = pltpu.async_copy(mean_hbm.at[col, :], mean_v, sem_b)
        cc = pltpu.async_copy(logvar_hbm.at[col, :], logvar_v, sem_c)
        cb.wait(); cc.wait(); ca.wait()

        row = jnp.zeros((VB,), jnp.int32)

        @pl.loop(0, NCH)
        def _(c):
            i0 = pl.multiple_of(c * CHUNK, CHUNK)

            @pl.loop(0, CHUNK // (2 * GRP * VB))
            def _(g):
                b0 = i0 + pl.multiple_of(g * 2 * GRP * VB, 2 * GRP * VB)
                sls = [pl.ds(b0 + v * VB, VB) for v in range(2 * GRP)]
                labs = [lab_v[0, sl] for sl in sls]
                ms = [plsc.load_gather(mean_v, [row, lab]) for lab in labs]
                lvs = [plsc.load_gather(logvar_v, [row, lab]) for lab in labs]
                for sl, m in zip(sls, ms):
                    om_v[0, sl] = m
                for sl, lv in zip(sls, lvs):
                    olv_v[0, sl] = lv

            cs = pl.ds(c * CHUNK, CHUNK)
            pltpu.async_copy(om_v.at[:, cs], out_m_hbm.at[col, cs], sem_a)
            pltpu.async_copy(olv_v.at[:, cs], out_lv_hbm.at[col, cs], sem_b)

        pltpu.make_async_copy(om_v, out_m_hbm.at[col, :], sem_a).wait()
        pltpu.make_async_copy(olv_v, out_lv_hbm.at[col, :], sem_b).wait()


@jax.jit
def kernel(labels, mean, log_var):
    mesh = plsc.VectorSubcoreMesh(core_axis_name="core",
                                  subcore_axis_name="subcore")
    cp = pltpu.CompilerParams(use_tc_tiling_on_sc=True)
    if "needs_layout_passes" in pltpu.CompilerParams.__dataclass_fields__:
        cp = dataclasses.replace(cp, needs_layout_passes=False)
    run = pl.kernel(
        _gather_kernel,
        out_type=(jax.ShapeDtypeStruct((F, B), jnp.float32),
                  jax.ShapeDtypeStruct((F, B), jnp.float32)),
        mesh=mesh,
        scratch_types=[
            pltpu.VMEM((1, B), jnp.int32),
            pltpu.VMEM((1, K), jnp.float32),
            pltpu.VMEM((1, K), jnp.float32),
            pltpu.VMEM((1, B), jnp.float32),
            pltpu.VMEM((1, B), jnp.float32),
            pltpu.SemaphoreType.DMA,
            pltpu.SemaphoreType.DMA,
            pltpu.SemaphoreType.DMA,
        ],
        compiler_params=cp,
    )
    om_t, olv_t = run(labels.astype(jnp.int32).T, mean, log_var)
    return om_t.T, olv_t.T


# speedup vs baseline: 1.0083x; 1.0083x over previous
import dataclasses

import jax
import jax.numpy as jnp
from jax import lax
from jax.experimental import pallas as pl
from jax.experimental.pallas import tpu as pltpu
from jax.experimental.pallas import tpu_sc as plsc

B = 16384
F = 26
K = 1000
VB = 16
GRP = 8
CHUNK = 16 * VB * GRP
NCH = B // CHUNK


def _gather_kernel(lab_hbm, mean_hbm, logvar_hbm, out_m_hbm, out_lv_hbm,
                   lab_v, mean_v, logvar_v, om_v, olv_v,
                   sem_a, sem_b, sem_c):
    wid = lax.axis_index("subcore") * 2 + lax.axis_index("core")

    @pl.when(wid < F)
    def _():
        col = pl.ds(wid, 1)
        ca = pltpu.async_copy(lab_hbm.at[col, :], lab_v, sem_a)
        cb = pltpu.async_copy(mean_hbm.at[col, :], mean_v, sem_b)
        cc = pltpu.async_copy(logvar_hbm.at[col, :], logvar_v, sem_c)
        cb.wait(); cc.wait(); ca.wait()

        row = jnp.zeros((VB,), jnp.int32)

        @pl.loop(0, NCH)
        def _(c):
            i0 = pl.multiple_of(c * CHUNK, CHUNK)

            @pl.loop(0, CHUNK // (GRP * VB))
            def _(g):
                b0 = i0 + pl.multiple_of(g * GRP * VB, GRP * VB)
                sls = [pl.ds(b0 + v * VB, VB) for v in range(GRP)]
                labs = [lab_v[0, sl] for sl in sls]
                ms = [plsc.load_gather(mean_v, [row, lab]) for lab in labs]
                lvs = [plsc.load_gather(logvar_v, [row, lab]) for lab in labs]
                for sl, m in zip(sls, ms):
                    om_v[0, sl] = m
                for sl, lv in zip(sls, lvs):
                    olv_v[0, sl] = lv

            cs = pl.ds(c * CHUNK, CHUNK)
            pltpu.async_copy(om_v.at[:, cs], out_m_hbm.at[col, cs], sem_a)
            pltpu.async_copy(olv_v.at[:, cs], out_lv_hbm.at[col, cs], sem_b)

        pltpu.make_async_copy(om_v, out_m_hbm.at[col, :], sem_a).wait()
        pltpu.make_async_copy(olv_v, out_lv_hbm.at[col, :], sem_b).wait()


@jax.jit
def kernel(labels, mean, log_var):
    mesh = plsc.VectorSubcoreMesh(core_axis_name="core",
                                  subcore_axis_name="subcore")
    cp = pltpu.CompilerParams(use_tc_tiling_on_sc=True)
    if "needs_layout_passes" in pltpu.CompilerParams.__dataclass_fields__:
        cp = dataclasses.replace(cp, needs_layout_passes=False)
    run = pl.kernel(
        _gather_kernel,
        out_type=(jax.ShapeDtypeStruct((F, B), jnp.float32),
                  jax.ShapeDtypeStruct((F, B), jnp.float32)),
        mesh=mesh,
        scratch_types=[
            pltpu.VMEM((1, B), jnp.int32),
            pltpu.VMEM((1, K), jnp.float32),
            pltpu.VMEM((1, K), jnp.float32),
            pltpu.VMEM((1, B), jnp.float32),
            pltpu.VMEM((1, B), jnp.float32),
            pltpu.SemaphoreType.DMA,
            pltpu.SemaphoreType.DMA,
            pltpu.SemaphoreType.DMA,
        ],
        compiler_params=cp,
    )
    om_t, olv_t = run(labels.astype(jnp.int32).T, mean, log_var)
    return om_t.T, olv_t.T
